# baseline (speedup 1.0000x reference)
import functools

import jax
import jax.numpy as jnp
from jax import lax
from jax.experimental import pallas as pl
from jax.experimental.pallas import tpu as pltpu
from jax.experimental.pallas import tpu_sc as plsc

N = 50000
E = 800000
G = 64

NC = 2
NS = 16
L = 16
NW = NC * NS
CH = 128
SLOW_C = 0
CHA = 132
CHB = 260
NCHMAX = max(CHA, CHB)
TCH = NS * (CHA + CHB) + NCHMAX
E_PAD = TCH * CH
E_USED = NS * (CHA + CHB) * CH
N2 = 50048
RPT = N2 // NS

BN = 3128
NBLK = N2 // BN


def _make_sc_agg(nfeat):
  F = 16
  mesh = plsc.VectorSubcoreMesh(
      core_axis_name="c", subcore_axis_name="s",
      num_cores=NC, num_subcores=NS)

  D = 4

  def body(*refs):
    feats = refs[:nfeat]
    srcp, dstp, eap, wvec, zf = refs[nfeat:nfeat + 5]
    outs = refs[nfeat + 5:2 * nfeat + 5]
    sc = refs[2 * nfeat + 5:]
    src_v, dst_v = sc[0], sc[1]
    rbufs = [sc[2 + b * nfeat:2 + (b + 1) * nfeat] for b in range(D)]
    eabs = sc[2 + D * nfeat:2 + D * nfeat + D]
    wv_buf = sc[2 + D * nfeat + D]
    aggs = sc[3 + D * nfeat + D:3 + D * nfeat + D + nfeat]
    sems = sc[3 + D * nfeat + D + nfeat:3 + D * nfeat + D + nfeat + D]
    ssems = sc[3 + D * nfeat + D + nfeat + D:]

    c = lax.axis_index("c")
    s = lax.axis_index("s")
    nch = jnp.where(c == SLOW_C, CHA, CHB)
    cb = jnp.where(c == SLOW_C, s * CHA, NS * CHA + s * CHB)
    pltpu.sync_copy(srcp.at[pl.ds(cb, NCHMAX)], src_v)
    pltpu.sync_copy(dstp.at[pl.ds(cb, NCHMAX)], dst_v)
    pltpu.sync_copy(wvec, wv_buf)
    for agg in aggs:
      pltpu.sync_copy(zf.at[s], agg.at[pl.ds(s * RPT, RPT)])
    wvreg = wv_buf[...]
    wb = [jnp.take_along_axis(wvreg, jnp.full((L,), k, jnp.int32), axis=0)
          for k in range(4)]
    iota = lax.broadcasted_iota(jnp.int32, (L,), 0)
    plsc.subcore_barrier()

    def issue(j, rset, eab, sem):
      for f, r in zip(feats, rset):
        pltpu.async_copy(f.at[src_v.at[j]], r, sem)
      pltpu.async_copy(eap.at[pl.ds((cb + j) * (3 * CH), 3 * CH)], eab, sem)

    def drain(j, rset, eab, sem):
      for f, r in zip(feats, rset):
        pltpu.make_async_copy(f.at[src_v.at[j]], r, sem).wait()
      pltpu.make_async_copy(eap.at[pl.ds((cb + j) * (3 * CH), 3 * CH)],
                            eab, sem).wait()

    def compute(j, rset, eab):
      iota3 = iota * 3
      for g in range(CH // L):
        ea = [plsc.load_gather(eab, [iota3 + (g * 3 * L + k)])
              for k in range(3)]
        ew = ea[0] * wb[0] + ea[1] * wb[1] + ea[2] * wb[2] + wb[3]
        for l in range(L):
          wbl = jnp.take_along_axis(ew, jnp.full((L,), l, jnp.int32), axis=0)
          r = g * L + l
          for rows in rset:
            rows[r] = rows[r] * wbl

    def scatter(j, rset, sem):
      for rows, agg in zip(rset, aggs):
        pltpu.async_copy(rows, agg.at[dst_v.at[j]], sem, add=True)

    def scatter_wait(j, rset, sem):
      for rows, agg in zip(rset, aggs):
        pltpu.make_async_copy(rows, agg.at[dst_v.at[j]], sem).wait()

    issue(0, rbufs[0], eabs[0], sems[0])
    issue(1, rbufs[1], eabs[1], sems[1])

    def group(g, carry):
      for d in range(D):
        j = D * g + d
        nd = (d + 2) % D

        @pl.when(jnp.logical_and(j >= 2, j + 2 < nch))
        def _():
          scatter_wait(j - 2, rbufs[nd], ssems[nd])
          issue(j + 2, rbufs[nd], eabs[nd], sems[nd])

        @pl.when(j < 2)
        def _():
          issue(j + 2, rbufs[nd], eabs[nd], sems[nd])

        drain(j, rbufs[d], eabs[d], sems[d])
        compute(j, rbufs[d], eabs[d])
        scatter(j, rbufs[d], ssems[d])
      return carry

    lax.fori_loop(0, nch // D, group, 0)
    for d in range(D):
      scatter_wait(nch - D + d, rbufs[d], ssems[d])
    plsc.subcore_barrier()
    for agg, out in zip(aggs, outs):
      pltpu.sync_copy(agg.at[pl.ds(s * RPT, RPT)],
                      out.at[c, pl.ds(s * RPT, RPT)])

  return pl.kernel(
      body,
      out_type=[jax.ShapeDtypeStruct((NC, N2, F), jnp.float32)
                for _ in range(nfeat)],
      mesh=mesh,
      scratch_types=(
          [pltpu.VMEM((NCHMAX, CH), jnp.int32),
           pltpu.VMEM((NCHMAX, CH), jnp.int32)]
          + [pltpu.VMEM((CH, F), jnp.float32)
             for _ in range(4 * nfeat)]
          + [pltpu.VMEM((3 * CH,), jnp.float32)
             for _ in range(4)]
          + [pltpu.VMEM((L,), jnp.float32)]
          + [pltpu.VMEM_SHARED((N2, F), jnp.float32)
             for _ in range(nfeat)]
          + [pltpu.SemaphoreType.DMA for _ in range(8)]),
      compiler_params=pltpu.CompilerParams(use_tc_tiling_on_sc=False,
                                           needs_layout_passes=False),
  )


_sc_agg1 = _make_sc_agg(1)


def _row_mask(i):
  rid = lax.broadcasted_iota(jnp.int32, (BN, 1), 0) + i * BN
  return (rid < N).astype(jnp.float32)


def _c1_body(agg_ref, x_ref, wa_ref, wr_ref, b_ref, t1_ref, st_ref, acc_ref):
  i = pl.program_id(0)
  a = agg_ref[0] + agg_ref[1]
  t = (jnp.dot(a, wa_ref[...], preferred_element_type=jnp.float32)
       + jnp.dot(x_ref[...], wr_ref[...], preferred_element_type=jnp.float32)
       + b_ref[...])
  t1_ref[...] = t
  ts = t * _row_mask(i)
  sums = jnp.concatenate(
      [jnp.sum(ts, axis=0, keepdims=True),
       jnp.sum(ts * ts, axis=0, keepdims=True)], axis=0)
  acc_ref[...] = jnp.where(i == 0, sums, acc_ref[...] + sums)

  @pl.when(i == NBLK - 1)
  def _():
    st_ref[...] = acc_ref[...]


def _bn_apply(t, st, g, b):
  mu = st[0:1, :] * (1.0 / N)
  var = st[1:2, :] * (1.0 / N) - mu * mu
  rstd = lax.rsqrt(var + 1e-5)
  return jnp.maximum((t - mu) * (rstd * g) + b, 0.0)


def _c2_body(t1_ref, st_ref, g_ref, b_ref, wc_ref, hplo_ref, hphi_ref, r2_ref):
  h1 = _bn_apply(t1_ref[...], st_ref[...], g_ref[...], b_ref[...])
  hp = jnp.dot(h1, wc_ref[...], preferred_element_type=jnp.float32)
  hplo_ref[...] = hp[:, :16]
  hphi_ref[...] = hp[:, 16:32]
  r2_ref[...] = hp[:, 32:]


def _e1_body(agglo_ref, agghi_ref, r2_ref, b_ref, t2_ref, st_ref, acc_ref):
  i = pl.program_id(0)
  agg = jnp.concatenate(
      [agglo_ref[0] + agglo_ref[1], agghi_ref[0] + agghi_ref[1]], axis=1)
  t = agg + r2_ref[...] + b_ref[...]
  t2_ref[...] = t
  ts = t * _row_mask(i)
  sums = jnp.concatenate(
      [jnp.sum(ts, axis=0, keepdims=True),
       jnp.sum(ts * ts, axis=0, keepdims=True)], axis=0)
  acc_ref[...] = jnp.where(i == 0, sums, acc_ref[...] + sums)

  @pl.when(i == NBLK - 1)
  def _():
    st_ref[...] = acc_ref[...]


def _e2_body(t2_ref, st_ref, g_ref, b_ref, bat_ref,
             l1w_ref, l1b_ref, l2w_ref, l2b_ref,
             out_ref, gs_ref, gc_ref):
  i = pl.program_id(0)
  h2 = _bn_apply(t2_ref[...], st_ref[...], g_ref[...], b_ref[...])
  gids = lax.broadcasted_iota(jnp.int32, (1, G), 1)
  oh = (bat_ref[...] == gids).astype(jnp.float32)
  gs = lax.dot_general(oh, h2, (((0,), (0,)), ((), ())),
                       preferred_element_type=jnp.float32)
  gc = lax.dot_general(oh, jnp.ones((BN, 1), jnp.float32),
                       (((0,), (0,)), ((), ())),
                       preferred_element_type=jnp.float32)
  gs_ref[...] = jnp.where(i == 0, gs, gs_ref[...] + gs)
  gc_ref[...] = jnp.where(i == 0, gc, gc_ref[...] + gc)

  @pl.when(i == NBLK - 1)
  def _():
    gx = gs_ref[...] / jnp.maximum(gc_ref[...], 1.0)
    z = jnp.maximum(
        jnp.dot(gx, l1w_ref[...], preferred_element_type=jnp.float32)
        + l1b_ref[...], 0.0)
    out_ref[...] = (
        jnp.dot(z, l2w_ref[...], preferred_element_type=jnp.float32)
        + l2b_ref[...])


_SEQ = pltpu.CompilerParams(dimension_semantics=("arbitrary",))


@jax.jit
def kernel(x, edge_index, edge_attr, batch, node_W, node_b, edge_W, edge_b,
           rel1_W, rel1_b, root1_W, bn1_g, bn1_b,
           rel2_W, rel2_b, root2_W, bn2_g, bn2_b,
           lin1_W, lin1_b, lin2_W, lin2_b):
  f32 = jnp.float32
  x = x.astype(f32)
  src = edge_index[0].astype(jnp.int32)
  dst = edge_index[1].astype(jnp.int32)

  x16 = jnp.pad(
      jnp.concatenate([x, jnp.ones((N, 1), f32), jnp.zeros((N, 6), f32)],
                      axis=1),
      ((0, N2 - N), (0, 0)))

  wa1 = jnp.concatenate(
      [node_W @ rel1_W, (node_b @ rel1_W)[None, :],
       jnp.zeros((6, 128), f32)], axis=0)
  wr1 = jnp.concatenate(
      [node_W @ root1_W, (node_b @ root1_W)[None, :],
       jnp.zeros((6, 128), f32)], axis=0)
  wcat = jnp.concatenate([rel2_W, root2_W], axis=1)

  pad = E_PAD - E
  srcp = jnp.concatenate([src, jnp.zeros((pad,), jnp.int32)]
                         ).reshape(TCH, CH)
  dstp = jnp.concatenate(
      [dst, N + (jnp.arange(pad, dtype=jnp.int32) % (N2 - N))]
      ).reshape(TCH, CH)
  eap = jnp.pad(edge_attr.astype(f32), ((0, pad), (0, 0))).reshape(-1)
  wvec = jnp.concatenate(
      [edge_W.reshape(3), edge_b.reshape(1), jnp.zeros((12,), f32)])

  z16 = jnp.zeros((NS, RPT, 16), f32)

  agg16, = _sc_agg1(x16, srcp, dstp, eap, wvec, z16)

  t1, st1 = pl.pallas_call(
      _c1_body,
      grid=(NBLK,),
      in_specs=[
          pl.BlockSpec((NC, BN, 16), lambda i: (0, i, 0)),
          pl.BlockSpec((BN, 16), lambda i: (i, 0)),
          pl.BlockSpec((16, 128), lambda i: (0, 0)),
          pl.BlockSpec((16, 128), lambda i: (0, 0)),
          pl.BlockSpec((1, 128), lambda i: (0, 0)),
      ],
      out_specs=[
          pl.BlockSpec((BN, 128), lambda i: (i, 0)),
          pl.BlockSpec((2, 128), lambda i: (0, 0)),
      ],
      out_shape=[
          jax.ShapeDtypeStruct((N2, 128), f32),
          jax.ShapeDtypeStruct((2, 128), f32),
      ],
      scratch_shapes=[pltpu.VMEM((2, 128), f32)],
      compiler_params=_SEQ,
  )(agg16, x16, wa1, wr1, rel1_b.reshape(1, 128))

  h1p_lo, h1p_hi, r2 = pl.pallas_call(
      _c2_body,
      grid=(NBLK,),
      in_specs=[
          pl.BlockSpec((BN, 128), lambda i: (i, 0)),
          pl.BlockSpec((2, 128), lambda i: (0, 0)),
          pl.BlockSpec((1, 128), lambda i: (0, 0)),
          pl.BlockSpec((1, 128), lambda i: (0, 0)),
          pl.BlockSpec((128, 64), lambda i: (0, 0)),
      ],
      out_specs=[
          pl.BlockSpec((BN, 16), lambda i: (i, 0)),
          pl.BlockSpec((BN, 16), lambda i: (i, 0)),
          pl.BlockSpec((BN, 32), lambda i: (i, 0)),
      ],
      out_shape=[
          jax.ShapeDtypeStruct((N2, 16), f32),
          jax.ShapeDtypeStruct((N2, 16), f32),
          jax.ShapeDtypeStruct((N2, 32), f32),
      ],
      compiler_params=_SEQ,
  )(t1, st1, bn1_g.reshape(1, 128), bn1_b.reshape(1, 128), wcat)

  agg32_lo, = _sc_agg1(h1p_lo, srcp, dstp, eap, wvec, z16)
  agg32_hi, = _sc_agg1(h1p_hi, srcp, dstp, eap, wvec, z16)

  t2, st2 = pl.pallas_call(
      _e1_body,
      grid=(NBLK,),
      in_specs=[
          pl.BlockSpec((NC, BN, 16), lambda i: (0, i, 0)),
          pl.BlockSpec((NC, BN, 16), lambda i: (0, i, 0)),
          pl.BlockSpec((BN, 32), lambda i: (i, 0)),
          pl.BlockSpec((1, 32), lambda i: (0, 0)),
      ],
      out_specs=[
          pl.BlockSpec((BN, 32), lambda i: (i, 0)),
          pl.BlockSpec((2, 32), lambda i: (0, 0)),
      ],
      out_shape=[
          jax.ShapeDtypeStruct((N2, 32), f32),
          jax.ShapeDtypeStruct((2, 32), f32),
      ],
      scratch_shapes=[pltpu.VMEM((2, 32), f32)],
      compiler_params=_SEQ,
  )(agg32_lo, agg32_hi, r2, rel2_b.reshape(1, 32))

  pred = pl.pallas_call(
      _e2_body,
      grid=(NBLK,),
      in_specs=[
          pl.BlockSpec((BN, 32), lambda i: (i, 0)),
          pl.BlockSpec((2, 32), lambda i: (0, 0)),
          pl.BlockSpec((1, 32), lambda i: (0, 0)),
          pl.BlockSpec((1, 32), lambda i: (0, 0)),
          pl.BlockSpec((BN, 1), lambda i: (i, 0)),
          pl.BlockSpec((32, 16), lambda i: (0, 0)),
          pl.BlockSpec((1, 16), lambda i: (0, 0)),
          pl.BlockSpec((16, 2), lambda i: (0, 0)),
          pl.BlockSpec((1, 2), lambda i: (0, 0)),
      ],
      out_specs=pl.BlockSpec((G, 2), lambda i: (0, 0)),
      out_shape=jax.ShapeDtypeStruct((G, 2), f32),
      scratch_shapes=[pltpu.VMEM((G, 32), f32), pltpu.VMEM((G, 1), f32)],
      compiler_params=_SEQ,
  )(t2, st2, bn2_g.reshape(1, 32), bn2_b.reshape(1, 32),
    jnp.concatenate([batch.astype(jnp.int32),
                     jnp.full((N2 - N,), G, jnp.int32)]).reshape(N2, 1),
    lin1_W, lin1_b.reshape(1, 16), lin2_W, lin2_b.reshape(1, 2))

  return pred


# speedup vs baseline: 3.9423x; 3.9423x over previous
import functools

import jax
import jax.numpy as jnp
from jax import lax
from jax.experimental import pallas as pl
from jax.experimental.pallas import tpu as pltpu
from jax.experimental.pallas import tpu_sc as plsc

N = 50000
E = 800000
G = 64

NC = 2
NS = 16
L = 16
NW = NC * NS
CH = 128
NCHUNK = 200
E_PAD = NW * NCHUNK * CH
N2 = 50048
RPT = N2 // NS

BN = 3128
NBLK = N2 // BN


def _make_sc_agg(nfeat):
  F = 16
  mesh = plsc.VectorSubcoreMesh(
      core_axis_name="c", subcore_axis_name="s",
      num_cores=NC, num_subcores=NS)

  D = 4

  NG = NCHUNK // D

  def body(*refs):
    feats = refs[:nfeat]
    srcp, dstp, ea0p, ea1p, ea2p, wvec, zf = refs[nfeat:nfeat + 7]
    outs = refs[nfeat + 7:2 * nfeat + 7]
    sc = refs[2 * nfeat + 7:]
    src_v, dst_v = sc[0], sc[1]
    rbufs = [sc[2 + b * nfeat:2 + (b + 1) * nfeat] for b in range(D)]
    eabs = sc[2 + D * nfeat:2 + D * nfeat + D]
    wv_buf = sc[2 + D * nfeat + D]
    aggs = sc[3 + D * nfeat + D:3 + D * nfeat + D + nfeat]
    sems = sc[3 + D * nfeat + D + nfeat:3 + D * nfeat + D + nfeat + D]
    ssems = sc[3 + D * nfeat + D + nfeat + D:]

    c = lax.axis_index("c")
    s = lax.axis_index("s")
    wid = c * NS + s
    pltpu.sync_copy(srcp.at[wid], src_v)
    pltpu.sync_copy(dstp.at[wid], dst_v)
    pltpu.sync_copy(wvec, wv_buf)
    for agg in aggs:
      pltpu.sync_copy(zf.at[s], agg.at[pl.ds(s * RPT, RPT)])
    wvreg = wv_buf[...]
    wb = [jnp.take_along_axis(wvreg, jnp.full((L,), k, jnp.int32), axis=0)
          for k in range(4)]
    plsc.subcore_barrier()

    def issue(j, rset, eab, sem):
      for f, r in zip(feats, rset):
        pltpu.async_copy(f.at[src_v.at[j]], r, sem)
      pltpu.async_copy(ea0p.at[wid, j], eab.at[pl.ds(0, CH)], sem)
      pltpu.async_copy(ea1p.at[wid, j], eab.at[pl.ds(CH, CH)], sem)
      pltpu.async_copy(ea2p.at[wid, j], eab.at[pl.ds(2 * CH, CH)], sem)

    def drain(j, rset, eab, sem):
      for f, r in zip(feats, rset):
        pltpu.make_async_copy(f.at[src_v.at[j]], r, sem).wait()
      pltpu.make_async_copy(ea0p.at[wid, j], eab.at[pl.ds(0, CH)], sem).wait()
      pltpu.make_async_copy(ea1p.at[wid, j], eab.at[pl.ds(CH, CH)], sem).wait()
      pltpu.make_async_copy(ea2p.at[wid, j], eab.at[pl.ds(2 * CH, CH)],
                            sem).wait()

    def compute(j, rset, eab):
      for g in range(CH // L):
        ew = (eab[pl.ds(g * L, L)] * wb[0]
              + eab[pl.ds(CH + g * L, L)] * wb[1]
              + eab[pl.ds(2 * CH + g * L, L)] * wb[2] + wb[3])
        for l in range(L):
          wbl = jnp.take_along_axis(ew, jnp.full((L,), l, jnp.int32), axis=0)
          r = g * L + l
          for rows in rset:
            rows[r] = rows[r] * wbl

    def scatter(j, rset, sem):
      for rows, agg in zip(rset, aggs):
        pltpu.async_copy(rows, agg.at[dst_v.at[j]], sem, add=True)

    def scatter_wait(j, rset, sem):
      for rows, agg in zip(rset, aggs):
        pltpu.make_async_copy(rows, agg.at[dst_v.at[j]], sem).wait()

    issue(0, rbufs[0], eabs[0], sems[0])
    issue(1, rbufs[1], eabs[1], sems[1])

    def group(g, carry):
      for d in range(D):
        j = D * g + d
        nd = (d + 2) % D

        @pl.when(jnp.logical_and(j >= 2, j + 2 < NCHUNK))
        def _():
          scatter_wait(j - 2, rbufs[nd], ssems[nd])
          issue(j + 2, rbufs[nd], eabs[nd], sems[nd])

        @pl.when(j < 2)
        def _():
          issue(j + 2, rbufs[nd], eabs[nd], sems[nd])

        drain(j, rbufs[d], eabs[d], sems[d])
        compute(j, rbufs[d], eabs[d])
        scatter(j, rbufs[d], ssems[d])
      return carry

    lax.fori_loop(0, NG, group, 0)
    for d in range(D):
      scatter_wait(NCHUNK - D + d, rbufs[d], ssems[d])
    plsc.subcore_barrier()
    for agg, out in zip(aggs, outs):
      pltpu.sync_copy(agg.at[pl.ds(s * RPT, RPT)],
                      out.at[c, pl.ds(s * RPT, RPT)])

  return pl.kernel(
      body,
      out_type=[jax.ShapeDtypeStruct((NC, N2, F), jnp.float32)
                for _ in range(nfeat)],
      mesh=mesh,
      scratch_types=(
          [pltpu.VMEM((NCHUNK, CH), jnp.int32),
           pltpu.VMEM((NCHUNK, CH), jnp.int32)]
          + [pltpu.VMEM((CH, F), jnp.float32)
             for _ in range(4 * nfeat)]
          + [pltpu.VMEM((3 * CH,), jnp.float32)
             for _ in range(4)]
          + [pltpu.VMEM((L,), jnp.float32)]
          + [pltpu.VMEM_SHARED((N2, F), jnp.float32)
             for _ in range(nfeat)]
          + [pltpu.SemaphoreType.DMA for _ in range(8)]),
      compiler_params=pltpu.CompilerParams(use_tc_tiling_on_sc=False),
  )


_sc_agg1 = _make_sc_agg(1)


def _row_mask(i):
  rid = lax.broadcasted_iota(jnp.int32, (BN, 1), 0) + i * BN
  return (rid < N).astype(jnp.float32)


def _c1_body(agg_ref, x_ref, wa_ref, wr_ref, b_ref, t1_ref, st_ref, acc_ref):
  i = pl.program_id(0)
  a = agg_ref[0] + agg_ref[1]
  t = (jnp.dot(a, wa_ref[...], preferred_element_type=jnp.float32)
       + jnp.dot(x_ref[...], wr_ref[...], preferred_element_type=jnp.float32)
       + b_ref[...])
  t1_ref[...] = t
  ts = t * _row_mask(i)
  sums = jnp.concatenate(
      [jnp.sum(ts, axis=0, keepdims=True),
       jnp.sum(ts * ts, axis=0, keepdims=True)], axis=0)
  acc_ref[...] = jnp.where(i == 0, sums, acc_ref[...] + sums)

  @pl.when(i == NBLK - 1)
  def _():
    st_ref[...] = acc_ref[...]


def _bn_apply(t, st, g, b):
  mu = st[0:1, :] * (1.0 / N)
  var = st[1:2, :] * (1.0 / N) - mu * mu
  rstd = lax.rsqrt(var + 1e-5)
  return jnp.maximum((t - mu) * (rstd * g) + b, 0.0)


def _c2_body(t1_ref, st_ref, g_ref, b_ref, wc_ref, hplo_ref, hphi_ref, r2_ref):
  h1 = _bn_apply(t1_ref[...], st_ref[...], g_ref[...], b_ref[...])
  hp = jnp.dot(h1, wc_ref[...], preferred_element_type=jnp.float32)
  hplo_ref[...] = hp[:, :16]
  hphi_ref[...] = hp[:, 16:32]
  r2_ref[...] = hp[:, 32:]


def _e1_body(agglo_ref, agghi_ref, r2_ref, b_ref, t2_ref, st_ref, acc_ref):
  i = pl.program_id(0)
  agg = jnp.concatenate(
      [agglo_ref[0] + agglo_ref[1], agghi_ref[0] + agghi_ref[1]], axis=1)
  t = agg + r2_ref[...] + b_ref[...]
  t2_ref[...] = t
  ts = t * _row_mask(i)
  sums = jnp.concatenate(
      [jnp.sum(ts, axis=0, keepdims=True),
       jnp.sum(ts * ts, axis=0, keepdims=True)], axis=0)
  acc_ref[...] = jnp.where(i == 0, sums, acc_ref[...] + sums)

  @pl.when(i == NBLK - 1)
  def _():
    st_ref[...] = acc_ref[...]


def _e2_body(t2_ref, st_ref, g_ref, b_ref, bat_ref,
             l1w_ref, l1b_ref, l2w_ref, l2b_ref,
             out_ref, gs_ref, gc_ref):
  i = pl.program_id(0)
  h2 = _bn_apply(t2_ref[...], st_ref[...], g_ref[...], b_ref[...])
  gids = lax.broadcasted_iota(jnp.int32, (1, G), 1)
  oh = (bat_ref[...] == gids).astype(jnp.float32)
  gs = lax.dot_general(oh, h2, (((0,), (0,)), ((), ())),
                       preferred_element_type=jnp.float32)
  gc = lax.dot_general(oh, jnp.ones((BN, 1), jnp.float32),
                       (((0,), (0,)), ((), ())),
                       preferred_element_type=jnp.float32)
  gs_ref[...] = jnp.where(i == 0, gs, gs_ref[...] + gs)
  gc_ref[...] = jnp.where(i == 0, gc, gc_ref[...] + gc)

  @pl.when(i == NBLK - 1)
  def _():
    gx = gs_ref[...] / jnp.maximum(gc_ref[...], 1.0)
    z = jnp.maximum(
        jnp.dot(gx, l1w_ref[...], preferred_element_type=jnp.float32)
        + l1b_ref[...], 0.0)
    out_ref[...] = (
        jnp.dot(z, l2w_ref[...], preferred_element_type=jnp.float32)
        + l2b_ref[...])


_SEQ = pltpu.CompilerParams(dimension_semantics=("arbitrary",))


@jax.jit
def kernel(x, edge_index, edge_attr, batch, node_W, node_b, edge_W, edge_b,
           rel1_W, rel1_b, root1_W, bn1_g, bn1_b,
           rel2_W, rel2_b, root2_W, bn2_g, bn2_b,
           lin1_W, lin1_b, lin2_W, lin2_b):
  f32 = jnp.float32
  x = x.astype(f32)
  src = edge_index[0].astype(jnp.int32)
  dst = edge_index[1].astype(jnp.int32)

  x16 = jnp.pad(
      jnp.concatenate([x, jnp.ones((N, 1), f32), jnp.zeros((N, 6), f32)],
                      axis=1),
      ((0, N2 - N), (0, 0)))

  wa1 = jnp.concatenate(
      [node_W @ rel1_W, (node_b @ rel1_W)[None, :],
       jnp.zeros((6, 128), f32)], axis=0)
  wr1 = jnp.concatenate(
      [node_W @ root1_W, (node_b @ root1_W)[None, :],
       jnp.zeros((6, 128), f32)], axis=0)
  wcat = jnp.concatenate([rel2_W, root2_W], axis=1)

  pad = E_PAD - E
  srcp = jnp.concatenate([src, jnp.zeros((pad,), jnp.int32)]
                         ).reshape(NW, NCHUNK, CH)
  dstp = jnp.concatenate(
      [dst, N + (jnp.arange(pad, dtype=jnp.int32) % (N2 - N))]
      ).reshape(NW, NCHUNK, CH)
  eac = edge_attr.astype(f32)
  ea0p, ea1p, ea2p = (
      jnp.concatenate([eac[:, k], jnp.zeros((pad,), f32)]
                      ).reshape(NW, NCHUNK, CH)
      for k in range(3))
  wvec = jnp.concatenate(
      [edge_W.reshape(3), edge_b.reshape(1), jnp.zeros((12,), f32)])

  z16 = jnp.zeros((NS, RPT, 16), f32)

  agg16, = _sc_agg1(x16, srcp, dstp, ea0p, ea1p, ea2p, wvec, z16)

  t1, st1 = pl.pallas_call(
      _c1_body,
      grid=(NBLK,),
      in_specs=[
          pl.BlockSpec((NC, BN, 16), lambda i: (0, i, 0)),
          pl.BlockSpec((BN, 16), lambda i: (i, 0)),
          pl.BlockSpec((16, 128), lambda i: (0, 0)),
          pl.BlockSpec((16, 128), lambda i: (0, 0)),
          pl.BlockSpec((1, 128), lambda i: (0, 0)),
      ],
      out_specs=[
          pl.BlockSpec((BN, 128), lambda i: (i, 0)),
          pl.BlockSpec((2, 128), lambda i: (0, 0)),
      ],
      out_shape=[
          jax.ShapeDtypeStruct((N2, 128), f32),
          jax.ShapeDtypeStruct((2, 128), f32),
      ],
      scratch_shapes=[pltpu.VMEM((2, 128), f32)],
      compiler_params=_SEQ,
  )(agg16, x16, wa1, wr1, rel1_b.reshape(1, 128))

  h1p_lo, h1p_hi, r2 = pl.pallas_call(
      _c2_body,
      grid=(NBLK,),
      in_specs=[
          pl.BlockSpec((BN, 128), lambda i: (i, 0)),
          pl.BlockSpec((2, 128), lambda i: (0, 0)),
          pl.BlockSpec((1, 128), lambda i: (0, 0)),
          pl.BlockSpec((1, 128), lambda i: (0, 0)),
          pl.BlockSpec((128, 64), lambda i: (0, 0)),
      ],
      out_specs=[
          pl.BlockSpec((BN, 16), lambda i: (i, 0)),
          pl.BlockSpec((BN, 16), lambda i: (i, 0)),
          pl.BlockSpec((BN, 32), lambda i: (i, 0)),
      ],
      out_shape=[
          jax.ShapeDtypeStruct((N2, 16), f32),
          jax.ShapeDtypeStruct((N2, 16), f32),
          jax.ShapeDtypeStruct((N2, 32), f32),
      ],
      compiler_params=_SEQ,
  )(t1, st1, bn1_g.reshape(1, 128), bn1_b.reshape(1, 128), wcat)

  agg32_lo, = _sc_agg1(h1p_lo, srcp, dstp, ea0p, ea1p, ea2p, wvec, z16)
  agg32_hi, = _sc_agg1(h1p_hi, srcp, dstp, ea0p, ea1p, ea2p, wvec, z16)

  t2, st2 = pl.pallas_call(
      _e1_body,
      grid=(NBLK,),
      in_specs=[
          pl.BlockSpec((NC, BN, 16), lambda i: (0, i, 0)),
          pl.BlockSpec((NC, BN, 16), lambda i: (0, i, 0)),
          pl.BlockSpec((BN, 32), lambda i: (i, 0)),
          pl.BlockSpec((1, 32), lambda i: (0, 0)),
      ],
      out_specs=[
          pl.BlockSpec((BN, 32), lambda i: (i, 0)),
          pl.BlockSpec((2, 32), lambda i: (0, 0)),
      ],
      out_shape=[
          jax.ShapeDtypeStruct((N2, 32), f32),
          jax.ShapeDtypeStruct((2, 32), f32),
      ],
      scratch_shapes=[pltpu.VMEM((2, 32), f32)],
      compiler_params=_SEQ,
  )(agg32_lo, agg32_hi, r2, rel2_b.reshape(1, 32))

  pred = pl.pallas_call(
      _e2_body,
      grid=(NBLK,),
      in_specs=[
          pl.BlockSpec((BN, 32), lambda i: (i, 0)),
          pl.BlockSpec((2, 32), lambda i: (0, 0)),
          pl.BlockSpec((1, 32), lambda i: (0, 0)),
          pl.BlockSpec((1, 32), lambda i: (0, 0)),
          pl.BlockSpec((BN, 1), lambda i: (i, 0)),
          pl.BlockSpec((32, 16), lambda i: (0, 0)),
          pl.BlockSpec((1, 16), lambda i: (0, 0)),
          pl.BlockSpec((16, 2), lambda i: (0, 0)),
          pl.BlockSpec((1, 2), lambda i: (0, 0)),
      ],
      out_specs=pl.BlockSpec((G, 2), lambda i: (0, 0)),
      out_shape=jax.ShapeDtypeStruct((G, 2), f32),
      scratch_shapes=[pltpu.VMEM((G, 32), f32), pltpu.VMEM((G, 1), f32)],
      compiler_params=_SEQ,
  )(t2, st2, bn2_g.reshape(1, 32), bn2_b.reshape(1, 32),
    jnp.concatenate([batch.astype(jnp.int32),
                     jnp.full((N2 - N,), G, jnp.int32)]).reshape(N2, 1),
    lin1_W, lin1_b.reshape(1, 16), lin2_W, lin2_b.reshape(1, 2))

  return pred


# speedup vs baseline: 4.7324x; 1.2004x over previous
import functools

import jax
import jax.numpy as jnp
from jax import lax
from jax.experimental import pallas as pl
from jax.experimental.pallas import tpu as pltpu
from jax.experimental.pallas import tpu_sc as plsc

N = 50000
E = 800000
G = 64

NC = 2
NS = 16
L = 16
NW = NC * NS
CH = 128
CHA = 132
CHB = 260
NCHMAX = max(CHA, CHB)
E_PAD = NS * (CHA + CHB) * CH
N2 = 50048
RPT = N2 // NS

BN = 3128
NBLK = N2 // BN


def _make_sc_agg(nfeat):
  F = 16
  mesh = plsc.VectorSubcoreMesh(
      core_axis_name="c", subcore_axis_name="s",
      num_cores=NC, num_subcores=NS)

  D = 4

  def body(*refs):
    feats = refs[:nfeat]
    (srcp0, srcp1, dstp0, dstp1, ea00, ea01, ea10, ea11, ea20, ea21,
     wvec, zf) = refs[nfeat:nfeat + 12]
    outs = refs[nfeat + 12:2 * nfeat + 12]
    sc = refs[2 * nfeat + 12:]
    src_v, dst_v = sc[0], sc[1]
    rbufs = [sc[2 + b * nfeat:2 + (b + 1) * nfeat] for b in range(D)]
    eabs = sc[2 + D * nfeat:2 + D * nfeat + D]
    wv_buf = sc[2 + D * nfeat + D]
    aggs = sc[3 + D * nfeat + D:3 + D * nfeat + D + nfeat]
    sems = sc[3 + D * nfeat + D + nfeat:3 + D * nfeat + D + nfeat + D]
    ssems = sc[3 + D * nfeat + D + nfeat + D:]

    c = lax.axis_index("c")
    s = lax.axis_index("s")
    pltpu.sync_copy(wvec, wv_buf)
    for agg in aggs:
      pltpu.sync_copy(zf.at[s], agg.at[pl.ds(s * RPT, RPT)])
    wvreg = wv_buf[...]
    wb = [jnp.take_along_axis(wvreg, jnp.full((L,), k, jnp.int32), axis=0)
          for k in range(4)]
    plsc.subcore_barrier()

    def compute(rset, eab):
      for g in range(CH // L):
        ew = (eab[pl.ds(g * L, L)] * wb[0]
              + eab[pl.ds(CH + g * L, L)] * wb[1]
              + eab[pl.ds(2 * CH + g * L, L)] * wb[2] + wb[3])
        for l in range(L):
          wbl = jnp.take_along_axis(ew, jnp.full((L,), l, jnp.int32), axis=0)
          r = g * L + l
          for rows in rset:
            rows[r] = rows[r] * wbl

    def run_pass(nchunk, *, srcp, dstp, eas):
      pltpu.sync_copy(srcp.at[s], src_v.at[pl.ds(0, nchunk)])
      pltpu.sync_copy(dstp.at[s], dst_v.at[pl.ds(0, nchunk)])

      def issue(j, rset, eab, sem):
        for f, r in zip(feats, rset):
          pltpu.async_copy(f.at[src_v.at[j]], r, sem)
        for k in range(3):
          pltpu.async_copy(eas[k].at[s, j], eab.at[pl.ds(k * CH, CH)], sem)

      def drain(j, rset, eab, sem):
        for f, r in zip(feats, rset):
          pltpu.make_async_copy(f.at[src_v.at[j]], r, sem).wait()
        for k in range(3):
          pltpu.make_async_copy(eas[k].at[s, j], eab.at[pl.ds(k * CH, CH)],
                                sem).wait()

      def scatter(j, rset, sem):
        for rows, agg in zip(rset, aggs):
          pltpu.async_copy(rows, agg.at[dst_v.at[j]], sem, add=True)

      def scatter_wait(j, rset, sem):
        for rows, agg in zip(rset, aggs):
          pltpu.make_async_copy(rows, agg.at[dst_v.at[j]], sem).wait()

      issue(0, rbufs[0], eabs[0], sems[0])
      issue(1, rbufs[1], eabs[1], sems[1])

      def group(g, carry):
        for d in range(D):
          j = D * g + d
          nd = (d + 2) % D

          @pl.when(jnp.logical_and(j >= 2, j + 2 < nchunk))
          def _():
            scatter_wait(j - 2, rbufs[nd], ssems[nd])
            issue(j + 2, rbufs[nd], eabs[nd], sems[nd])

          @pl.when(j < 2)
          def _():
            issue(j + 2, rbufs[nd], eabs[nd], sems[nd])

          drain(j, rbufs[d], eabs[d], sems[d])
          compute(rbufs[d], eabs[d])
          scatter(j, rbufs[d], ssems[d])
        return carry

      lax.fori_loop(0, nchunk // D, group, 0)
      for d in range(D):
        scatter_wait(nchunk - D + d, rbufs[d], ssems[d])

    @pl.when(c == 0)
    def _():
      run_pass(CHA, srcp=srcp0, dstp=dstp0, eas=(ea00, ea10, ea20))

    @pl.when(c == 1)
    def _():
      run_pass(CHB, srcp=srcp1, dstp=dstp1, eas=(ea01, ea11, ea21))

    plsc.subcore_barrier()
    for agg, out in zip(aggs, outs):
      pltpu.sync_copy(agg.at[pl.ds(s * RPT, RPT)],
                      out.at[c, pl.ds(s * RPT, RPT)])

  return pl.kernel(
      body,
      out_type=[jax.ShapeDtypeStruct((NC, N2, F), jnp.float32)
                for _ in range(nfeat)],
      mesh=mesh,
      scratch_types=(
          [pltpu.VMEM((NCHMAX, CH), jnp.int32),
           pltpu.VMEM((NCHMAX, CH), jnp.int32)]
          + [pltpu.VMEM((CH, F), jnp.float32)
             for _ in range(4 * nfeat)]
          + [pltpu.VMEM((3 * CH,), jnp.float32)
             for _ in range(4)]
          + [pltpu.VMEM((L,), jnp.float32)]
          + [pltpu.VMEM_SHARED((N2, F), jnp.float32)
             for _ in range(nfeat)]
          + [pltpu.SemaphoreType.DMA for _ in range(8)]),
      compiler_params=pltpu.CompilerParams(use_tc_tiling_on_sc=False),
  )


_sc_agg1 = _make_sc_agg(1)


def _row_mask(i):
  rid = lax.broadcasted_iota(jnp.int32, (BN, 1), 0) + i * BN
  return (rid < N).astype(jnp.float32)


def _c1_body(agg_ref, x_ref, wa_ref, wr_ref, b_ref, t1_ref, st_ref, acc_ref):
  i = pl.program_id(0)
  a = agg_ref[0] + agg_ref[1]
  t = (jnp.dot(a, wa_ref[...], preferred_element_type=jnp.float32)
       + jnp.dot(x_ref[...], wr_ref[...], preferred_element_type=jnp.float32)
       + b_ref[...])
  t1_ref[...] = t
  ts = t * _row_mask(i)
  sums = jnp.concatenate(
      [jnp.sum(ts, axis=0, keepdims=True),
       jnp.sum(ts * ts, axis=0, keepdims=True)], axis=0)
  acc_ref[...] = jnp.where(i == 0, sums, acc_ref[...] + sums)

  @pl.when(i == NBLK - 1)
  def _():
    st_ref[...] = acc_ref[...]


def _bn_apply(t, st, g, b):
  mu = st[0:1, :] * (1.0 / N)
  var = st[1:2, :] * (1.0 / N) - mu * mu
  rstd = lax.rsqrt(var + 1e-5)
  return jnp.maximum((t - mu) * (rstd * g) + b, 0.0)


def _c2_body(t1_ref, st_ref, g_ref, b_ref, wc_ref, hplo_ref, hphi_ref, r2_ref):
  h1 = _bn_apply(t1_ref[...], st_ref[...], g_ref[...], b_ref[...])
  hp = jnp.dot(h1, wc_ref[...], preferred_element_type=jnp.float32)
  hplo_ref[...] = hp[:, :16]
  hphi_ref[...] = hp[:, 16:32]
  r2_ref[...] = hp[:, 32:]


def _e1_body(agglo_ref, agghi_ref, r2_ref, b_ref, t2_ref, st_ref, acc_ref):
  i = pl.program_id(0)
  agg = jnp.concatenate(
      [agglo_ref[0] + agglo_ref[1], agghi_ref[0] + agghi_ref[1]], axis=1)
  t = agg + r2_ref[...] + b_ref[...]
  t2_ref[...] = t
  ts = t * _row_mask(i)
  sums = jnp.concatenate(
      [jnp.sum(ts, axis=0, keepdims=True),
       jnp.sum(ts * ts, axis=0, keepdims=True)], axis=0)
  acc_ref[...] = jnp.where(i == 0, sums, acc_ref[...] + sums)

  @pl.when(i == NBLK - 1)
  def _():
    st_ref[...] = acc_ref[...]


def _e2_body(t2_ref, st_ref, g_ref, b_ref, bat_ref,
             l1w_ref, l1b_ref, l2w_ref, l2b_ref,
             out_ref, gs_ref, gc_ref):
  i = pl.program_id(0)
  h2 = _bn_apply(t2_ref[...], st_ref[...], g_ref[...], b_ref[...])
  gids = lax.broadcasted_iota(jnp.int32, (1, G), 1)
  oh = (bat_ref[...] == gids).astype(jnp.float32)
  gs = lax.dot_general(oh, h2, (((0,), (0,)), ((), ())),
                       preferred_element_type=jnp.float32)
  gc = lax.dot_general(oh, jnp.ones((BN, 1), jnp.float32),
                       (((0,), (0,)), ((), ())),
                       preferred_element_type=jnp.float32)
  gs_ref[...] = jnp.where(i == 0, gs, gs_ref[...] + gs)
  gc_ref[...] = jnp.where(i == 0, gc, gc_ref[...] + gc)

  @pl.when(i == NBLK - 1)
  def _():
    gx = gs_ref[...] / jnp.maximum(gc_ref[...], 1.0)
    z = jnp.maximum(
        jnp.dot(gx, l1w_ref[...], preferred_element_type=jnp.float32)
        + l1b_ref[...], 0.0)
    out_ref[...] = (
        jnp.dot(z, l2w_ref[...], preferred_element_type=jnp.float32)
        + l2b_ref[...])


_SEQ = pltpu.CompilerParams(dimension_semantics=("arbitrary",))


@jax.jit
def kernel(x, edge_index, edge_attr, batch, node_W, node_b, edge_W, edge_b,
           rel1_W, rel1_b, root1_W, bn1_g, bn1_b,
           rel2_W, rel2_b, root2_W, bn2_g, bn2_b,
           lin1_W, lin1_b, lin2_W, lin2_b):
  f32 = jnp.float32
  x = x.astype(f32)
  src = edge_index[0].astype(jnp.int32)
  dst = edge_index[1].astype(jnp.int32)

  x16 = jnp.pad(
      jnp.concatenate([x, jnp.ones((N, 1), f32), jnp.zeros((N, 6), f32)],
                      axis=1),
      ((0, N2 - N), (0, 0)))

  wa1 = jnp.concatenate(
      [node_W @ rel1_W, (node_b @ rel1_W)[None, :],
       jnp.zeros((6, 128), f32)], axis=0)
  wr1 = jnp.concatenate(
      [node_W @ root1_W, (node_b @ root1_W)[None, :],
       jnp.zeros((6, 128), f32)], axis=0)
  wcat = jnp.concatenate([rel2_W, root2_W], axis=1)

  pad = E_PAD - E
  E0 = NS * CHA * CH

  def _split(flat):
    return (flat[:E0].reshape(NS, CHA, CH),
            flat[E0:].reshape(NS, CHB, CH))

  srcp0, srcp1 = _split(
      jnp.concatenate([src, jnp.zeros((pad,), jnp.int32)]))
  dstp0, dstp1 = _split(jnp.concatenate(
      [dst, N + (jnp.arange(pad, dtype=jnp.int32) % (N2 - N))]))
  eac = edge_attr.astype(f32)
  (ea00, ea01), (ea10, ea11), (ea20, ea21) = (
      _split(jnp.concatenate([eac[:, k], jnp.zeros((pad,), f32)]))
      for k in range(3))
  wvec = jnp.concatenate(
      [edge_W.reshape(3), edge_b.reshape(1), jnp.zeros((12,), f32)])
  eargs = (srcp0, srcp1, dstp0, dstp1, ea00, ea01, ea10, ea11, ea20, ea21)

  z16 = jnp.zeros((NS, RPT, 16), f32)

  agg16, = _sc_agg1(x16, *eargs, wvec, z16)

  t1, st1 = pl.pallas_call(
      _c1_body,
      grid=(NBLK,),
      in_specs=[
          pl.BlockSpec((NC, BN, 16), lambda i: (0, i, 0)),
          pl.BlockSpec((BN, 16), lambda i: (i, 0)),
          pl.BlockSpec((16, 128), lambda i: (0, 0)),
          pl.BlockSpec((16, 128), lambda i: (0, 0)),
          pl.BlockSpec((1, 128), lambda i: (0, 0)),
      ],
      out_specs=[
          pl.BlockSpec((BN, 128), lambda i: (i, 0)),
          pl.BlockSpec((2, 128), lambda i: (0, 0)),
      ],
      out_shape=[
          jax.ShapeDtypeStruct((N2, 128), f32),
          jax.ShapeDtypeStruct((2, 128), f32),
      ],
      scratch_shapes=[pltpu.VMEM((2, 128), f32)],
      compiler_params=_SEQ,
  )(agg16, x16, wa1, wr1, rel1_b.reshape(1, 128))

  h1p_lo, h1p_hi, r2 = pl.pallas_call(
      _c2_body,
      grid=(NBLK,),
      in_specs=[
          pl.BlockSpec((BN, 128), lambda i: (i, 0)),
          pl.BlockSpec((2, 128), lambda i: (0, 0)),
          pl.BlockSpec((1, 128), lambda i: (0, 0)),
          pl.BlockSpec((1, 128), lambda i: (0, 0)),
          pl.BlockSpec((128, 64), lambda i: (0, 0)),
      ],
      out_specs=[
          pl.BlockSpec((BN, 16), lambda i: (i, 0)),
          pl.BlockSpec((BN, 16), lambda i: (i, 0)),
          pl.BlockSpec((BN, 32), lambda i: (i, 0)),
      ],
      out_shape=[
          jax.ShapeDtypeStruct((N2, 16), f32),
          jax.ShapeDtypeStruct((N2, 16), f32),
          jax.ShapeDtypeStruct((N2, 32), f32),
      ],
      compiler_params=_SEQ,
  )(t1, st1, bn1_g.reshape(1, 128), bn1_b.reshape(1, 128), wcat)

  agg32_lo, = _sc_agg1(h1p_lo, *eargs, wvec, z16)
  agg32_hi, = _sc_agg1(h1p_hi, *eargs, wvec, z16)

  t2, st2 = pl.pallas_call(
      _e1_body,
      grid=(NBLK,),
      in_specs=[
          pl.BlockSpec((NC, BN, 16), lambda i: (0, i, 0)),
          pl.BlockSpec((NC, BN, 16), lambda i: (0, i, 0)),
          pl.BlockSpec((BN, 32), lambda i: (i, 0)),
          pl.BlockSpec((1, 32), lambda i: (0, 0)),
      ],
      out_specs=[
          pl.BlockSpec((BN, 32), lambda i: (i, 0)),
          pl.BlockSpec((2, 32), lambda i: (0, 0)),
      ],
      out_shape=[
          jax.ShapeDtypeStruct((N2, 32), f32),
          jax.ShapeDtypeStruct((2, 32), f32),
      ],
      scratch_shapes=[pltpu.VMEM((2, 32), f32)],
      compiler_params=_SEQ,
  )(agg32_lo, agg32_hi, r2, rel2_b.reshape(1, 32))

  pred = pl.pallas_call(
      _e2_body,
      grid=(NBLK,),
      in_specs=[
          pl.BlockSpec((BN, 32), lambda i: (i, 0)),
          pl.BlockSpec((2, 32), lambda i: (0, 0)),
          pl.BlockSpec((1, 32), lambda i: (0, 0)),
          pl.BlockSpec((1, 32), lambda i: (0, 0)),
          pl.BlockSpec((BN, 1), lambda i: (i, 0)),
          pl.BlockSpec((32, 16), lambda i: (0, 0)),
          pl.BlockSpec((1, 16), lambda i: (0, 0)),
          pl.BlockSpec((16, 2), lambda i: (0, 0)),
          pl.BlockSpec((1, 2), lambda i: (0, 0)),
      ],
      out_specs=pl.BlockSpec((G, 2), lambda i: (0, 0)),
      out_shape=jax.ShapeDtypeStruct((G, 2), f32),
      scratch_shapes=[pltpu.VMEM((G, 32), f32), pltpu.VMEM((G, 1), f32)],
      compiler_params=_SEQ,
  )(t2, st2, bn2_g.reshape(1, 32), bn2_b.reshape(1, 32),
    jnp.concatenate([batch.astype(jnp.int32),
                     jnp.full((N2 - N,), G, jnp.int32)]).reshape(N2, 1),
    lin1_W, lin1_b.reshape(1, 16), lin2_W, lin2_b.reshape(1, 2))

  return pred
